# Initial kernel scaffold; baseline (speedup 1.0000x reference)
import jax
import jax.numpy as jnp
from jax.experimental import pallas as pl


def kernel(deg_idx, edge_index, batch, emb, Wl0, bl0, Wr0, g0, be0, Wl1, bl1, Wr1, g1, be1, Wa, ba, Wo, bo):
    raise NotImplementedError("write your pallas kernel here")


# speedup vs baseline: 3.9422x; 3.9422x over previous
import functools

import jax
import jax.numpy as jnp
from jax import lax
from jax.experimental import pallas as pl
from jax.experimental.pallas import tpu as pltpu
from jax.experimental.pallas import tpu_sc as plsc

_N = 10000
_E = 320000
_D = 128
_C = 8
_B = 8
_NV = 257

_NC = 2
_NS = 16
_NW = _NC * _NS
_K = 128
_CH = 79
_EPAD = _NW * _CH * _K
_NROWS = 10240
_RPT = _NROWS // _NS
_TRASH = _N


def _segsum_body(x_hbm, src_hbm, dst_hbm, sum_hbm, cnt_hbm,
                 sidx, didx, rows, ones, zb1, zbuf, agg_sh, cnt_sh, sem):
    c = lax.axis_index("c")
    s = lax.axis_index("s")
    wid = s * _NC + c

    zero16 = jnp.zeros((16,), jnp.float32)
    one16 = jnp.ones((16,), jnp.float32)
    for r in range(16):
        for q in range(8):
            zbuf[r, pl.ds(q * 16, 16)] = zero16
    for q in range(_RPT // 16):
        zb1[pl.ds(q * 16, 16)] = zero16
    for q in range(_K // 16):
        ones[pl.ds(q * 16, 16)] = one16

    r0 = s * _RPT

    @pl.loop(0, _RPT // 16)
    def _zero(i):
        pltpu.sync_copy(zbuf, agg_sh.at[pl.ds(r0 + i * 16, 16)])

    pltpu.sync_copy(zb1, cnt_sh.at[pl.ds(r0, _RPT)])
    plsc.subcore_barrier()

    base = wid * (_CH * _K)

    @pl.loop(0, _CH)
    def _edges(j):
        off = base + j * _K
        pltpu.sync_copy(src_hbm.at[pl.ds(off, _K)], sidx.at[0])
        pltpu.sync_copy(dst_hbm.at[pl.ds(off, _K)], didx.at[0])
        pltpu.async_copy(x_hbm.at[sidx.at[0]], rows, sem).wait()
        pltpu.sync_copy(rows, agg_sh.at[didx.at[0]], add=True)
        pltpu.sync_copy(ones, cnt_sh.at[didx.at[0]], add=True)

    plsc.subcore_barrier()

    pltpu.sync_copy(agg_sh.at[pl.ds(r0, _RPT)], sum_hbm.at[c, pl.ds(r0, _RPT)])
    pltpu.sync_copy(cnt_sh.at[pl.ds(r0, _RPT)], cnt_hbm.at[c, pl.ds(r0, _RPT)])


@functools.cache
def _segsum_call():
    return pl.kernel(
        _segsum_body,
        out_type=(
            jax.ShapeDtypeStruct((_NC, _NROWS, _D), jnp.float32),
            jax.ShapeDtypeStruct((_NC, _NROWS), jnp.float32),
        ),
        mesh=plsc.VectorSubcoreMesh(core_axis_name="c", subcore_axis_name="s"),
        scratch_types=[
            pltpu.VMEM((1, _K), jnp.int32),
            pltpu.VMEM((1, _K), jnp.int32),
            pltpu.VMEM((_K, _D), jnp.float32),
            pltpu.VMEM((_K,), jnp.float32),
            pltpu.VMEM((_RPT,), jnp.float32),
            pltpu.VMEM((16, _D), jnp.float32),
            pltpu.VMEM_SHARED((_NROWS, _D), jnp.float32),
            pltpu.VMEM_SHARED((_NROWS,), jnp.float32),
            pltpu.SemaphoreType.DMA,
        ],
    )


def _mmT(a, b):
    return lax.dot_general(a, b, (((1,), (1,)), ((), ())),
                           preferred_element_type=jnp.float32)


def _embed_tc(deg_ref, emb_ref, out_ref):
    deg = deg_ref[...]
    iota = lax.broadcasted_iota(jnp.int32, (_N, _NV), 1)
    oh = jnp.where(iota == deg, 1.0, 0.0)
    out_ref[...] = lax.dot_general(oh, emb_ref[...], (((1,), (0,)), ((), ())),
                                   preferred_element_type=jnp.float32)


def _layer_tc(x_ref, parts_ref, cnts_ref, wl_ref, bl_ref, wr_ref, g_ref,
              be_ref, out_ref):
    agg = parts_ref[0, :_N, :] + parts_ref[1, :_N, :]
    cnt = cnts_ref[0, :_N, :] + cnts_ref[1, :_N, :]
    aggm = agg * (1.0 / jnp.maximum(cnt, 1.0))
    h = _mmT(aggm, wl_ref[...]) + _mmT(x_ref[...], wr_ref[...]) + bl_ref[...]
    mean = jnp.mean(h, axis=0, keepdims=True)
    d = h - mean
    var = jnp.mean(d * d, axis=0, keepdims=True)
    y = d * lax.rsqrt(var + 1e-5) * g_ref[...] + be_ref[...]
    out_ref[...] = jnp.maximum(y, 0.0)


def _pool_tc(x_ref, batch_ref, wa_ref, ba_ref, wo_ref, bo_ref, out_ref):
    b = pl.program_id(0)
    x = x_ref[...]
    scores = _mmT(x, wa_ref[...]) + ba_ref[...]
    mask = batch_ref[...] == b
    s_i = jnp.where(mask, scores, -1e9)
    m = jnp.max(s_i, axis=0, keepdims=True)
    e = jnp.where(mask, jnp.exp(s_i - m), 0.0)
    denom = jnp.sum(e, axis=0, keepdims=True)
    w = e * (1.0 / jnp.maximum(denom, 1e-30))
    cvec = lax.dot_general(w, x, (((0,), (0,)), ((), ())),
                           preferred_element_type=jnp.float32)
    out_ref[0] = _mmT(cvec, wo_ref[...]) + bo_ref[...]


def _embed_call(deg2, emb):
    return pl.pallas_call(
        _embed_tc,
        out_shape=jax.ShapeDtypeStruct((_N, _D), jnp.float32),
    )(deg2, emb)


def _layer_call(x, parts, cnts, wl, bl, wr, g, be):
    return pl.pallas_call(
        _layer_tc,
        out_shape=jax.ShapeDtypeStruct((_N, _D), jnp.float32),
    )(x, parts, cnts, wl, bl, wr, g, be)


def _pool_call(x, batch2, wa, ba, wo, bo):
    return pl.pallas_call(
        _pool_tc,
        grid=(_B,),
        in_specs=[
            pl.BlockSpec((_N, _D), lambda b: (0, 0)),
            pl.BlockSpec((_N, 1), lambda b: (0, 0)),
            pl.BlockSpec((_C, _D), lambda b: (0, 0)),
            pl.BlockSpec((1, _C), lambda b: (0, 0)),
            pl.BlockSpec((_D, _D), lambda b: (0, 0)),
            pl.BlockSpec((1, _D), lambda b: (0, 0)),
        ],
        out_specs=pl.BlockSpec((1, _C, _D), lambda b: (b, 0, 0)),
        out_shape=jax.ShapeDtypeStruct((_B, _C, _D), jnp.float32),
    )(x, batch2, wa, ba, wo, bo)


def kernel(deg_idx, edge_index, batch, emb, Wl0, bl0, Wr0, g0, be0,
           Wl1, bl1, Wr1, g1, be1, Wa, ba, Wo, bo):
    src = edge_index[0].astype(jnp.int32)
    dst = edge_index[1].astype(jnp.int32)
    npad = _EPAD - _E
    src_p = jnp.concatenate([src, jnp.zeros((npad,), jnp.int32)])
    dst_p = jnp.concatenate([dst, jnp.full((npad,), _TRASH, jnp.int32)])

    deg2 = deg_idx.astype(jnp.int32).reshape(_N, 1)
    batch2 = batch.astype(jnp.int32).reshape(_N, 1)
    bl0r = bl0.reshape(1, _D)
    g0r = g0.reshape(1, _D)
    be0r = be0.reshape(1, _D)
    bl1r = bl1.reshape(1, _D)
    g1r = g1.reshape(1, _D)
    be1r = be1.reshape(1, _D)
    bar = ba.reshape(1, _C)
    bor = bo.reshape(1, _D)

    segsum = _segsum_call()

    x0 = _embed_call(deg2, emb)
    parts0, cnt0 = segsum(x0, src_p, dst_p)
    cnts0 = cnt0[:, :, None]
    x1 = _layer_call(x0, parts0, cnts0, Wl0, bl0r, Wr0, g0r, be0r)
    parts1, cnt1 = segsum(x1, src_p, dst_p)
    cnts1 = cnt1[:, :, None]
    x2 = _layer_call(x1, parts1, cnts1, Wl1, bl1r, Wr1, g1r, be1r)
    return _pool_call(x2, batch2, Wa, bar, Wo, bor)
